# baseline (speedup 1.0000x reference)
import functools

import jax
import jax.numpy as jnp
from jax import lax
from jax.experimental import pallas as pl
from jax.experimental.pallas import tpu as pltpu
from jax.experimental.pallas import tpu_sc as plsc

_E = 1024
_D = 64
_R = 32768
_TILE = 1024
_NT = _R // _TILE
_DECAY = 0.99
_EPS = 1e-05
_COMMIT = 0.25


def _stats_body(z_ref, w_ref, ecs_ref, ees_ref,
                idx_ref, uw_ref, loss_ref,
                cnt_acc, sum_acc, z2_acc, wneg_acc, wh2_acc):
    step = pl.program_id(0)

    @pl.when(step == 0)
    def _init():
        cnt_acc[...] = jnp.zeros_like(cnt_acc)
        sum_acc[...] = jnp.zeros_like(sum_acc)
        z2_acc[...] = jnp.zeros_like(z2_acc)
        w0 = w_ref[...]
        wneg_acc[...] = -w0
        wh2_acc[...] = 0.5 * jnp.sum(w0 * w0, axis=1, keepdims=True)

    rowid = lax.broadcasted_iota(jnp.int32, (_E, _TILE), 0)
    wneg = wneg_acc[...]
    wh2 = wh2_acc[...]
    for t in range(4):
        zT = z_ref[0, :, t, 0, :]
        distT = (lax.dot_general(wneg, zT, (((1,), (0,)), ((), ())))
                 + wh2)
        idx = jnp.argmin(distT, axis=0).astype(jnp.int32)
        idx_ref[0, t, :] = idx

        onehotT = (rowid == idx[None, :]).astype(jnp.float32)
        cnt_acc[...] += jnp.sum(onehotT, axis=1, keepdims=True)
        sum_acc[...] += lax.dot_general(onehotT, zT, (((1,), (1,)), ((), ())))
        z2_acc[...] += jnp.sum(zT * zT).reshape(1, 1)

    @pl.when(step == _NT // 4 - 1)
    def _fin():
        cnt = cnt_acc[...]
        new_cs = _DECAY * ecs_ref[...] + (1.0 - _DECAY) * cnt
        new_es = _DECAY * ees_ref[...] + (1.0 - _DECAY) * sum_acc[...]
        n = jnp.sum(new_cs)
        cs = (new_cs + _EPS) / (n + _E * _EPS) * n
        w_new = new_es / cs
        uw_ref[...] = jnp.concatenate(
            [w_new, jnp.zeros((_E, 128 - _D), jnp.float32)], axis=1)
        s_dot_w = jnp.sum(sum_acc[...] * w_new)
        c_w2 = jnp.sum(cnt * jnp.sum(w_new * w_new, axis=1, keepdims=True))
        total = z2_acc[...] - 2.0 * s_dot_w + c_w2
        loss_ref[...] = _COMMIT * total / float(_R * _D)


def _run_stats(z4, embedding_weight, ecs_col, ema_embed_sum):
    return pl.pallas_call(
        _stats_body,
        grid=(_NT // 4,),
        in_specs=[
            pl.BlockSpec((1, _D, 4, 1, _TILE), lambda i: (i // 2, 0, i % 2, 0, 0)),
            pl.BlockSpec((_E, _D), lambda i: (0, 0)),
            pl.BlockSpec((_E, 1), lambda i: (0, 0)),
            pl.BlockSpec((_E, _D), lambda i: (0, 0)),
        ],
        out_specs=[
            pl.BlockSpec((1, 4, _TILE), lambda i: (i, 0, 0)),
            pl.BlockSpec((_E, 128), lambda i: (0, 0)),
            pl.BlockSpec((1, 1), lambda i: (0, 0)),
        ],
        out_shape=[
            jax.ShapeDtypeStruct((_NT // 4, 4, _TILE), jnp.int32),
            jax.ShapeDtypeStruct((_E, 128), jnp.float32),
            jax.ShapeDtypeStruct((1, 1), jnp.float32),
        ],
        scratch_shapes=[
            pltpu.VMEM((_E, 1), jnp.float32),
            pltpu.VMEM((_E, _D), jnp.float32),
            pltpu.VMEM((1, 1), jnp.float32),
            pltpu.VMEM((_E, _D), jnp.float32),
            pltpu.VMEM((_E, 1), jnp.float32),
        ],
    )(z4, embedding_weight, ecs_col, ema_embed_sum)


def _run_sc_gather(table, idx_flat):
    info = plsc.get_sparse_core_info()
    nw = info.num_cores * info.num_subcores
    b_per_w = _R // nw
    n_chunks = b_per_w // 128
    mesh = plsc.VectorSubcoreMesh(core_axis_name="c", subcore_axis_name="s")

    @functools.partial(
        pl.kernel, mesh=mesh,
        out_type=jax.ShapeDtypeStruct((_R, 128), jnp.float32),
        scratch_types=[
            pltpu.VMEM((n_chunks, 128), jnp.int32),
            pltpu.VMEM((2 * 2 * 128, 128), jnp.float32),
            pltpu.SemaphoreType.DMA,
            pltpu.SemaphoreType.DMA,
        ],
    )
    def k(table_hbm, idx_hbm, out_hbm, idx_v, rows_v, gsem, ssem):
        wid = lax.axis_index("s") * info.num_cores + lax.axis_index("c")
        base = wid * b_per_w
        pltpu.sync_copy(idx_hbm.at[pl.ds(wid * n_chunks, n_chunks)], idx_v)
        stores = [None, None]
        for r in range(4):
            buf = r % 2
            if stores[buf] is not None:
                stores[buf].wait()
            gathers = []
            for j in range(2):
                gathers.append(pltpu.async_copy(
                    table_hbm.at[idx_v.at[2 * r + j]],
                    rows_v.at[pl.ds((2 * buf + j) * 128, 128)], gsem))
            for g in gathers:
                g.wait()
            stores[buf] = pltpu.async_copy(
                rows_v.at[pl.ds(2 * buf * 128, 256)],
                out_hbm.at[pl.ds(base + r * 256, 256)], ssem)
        for s in stores:
            s.wait()

    return k(table, idx_flat)


def _gather_body(idx_ref, w_ref, out_ref):
    idx = idx_ref[0, 0, :]
    rowid = lax.broadcasted_iota(jnp.int32, (_E, _TILE), 0)
    onehotT = (rowid == idx[None, :]).astype(jnp.float32)
    out_ref[...] = lax.dot_general(onehotT, w_ref[...], (((0,), (0,)), ((), ())))


def _run_gather(idx3, updated_weight):
    return pl.pallas_call(
        _gather_body,
        grid=(_NT // 4,),
        in_specs=[
            pl.BlockSpec((1, 1, _TILE), lambda i: (i, 0, 0)),
            pl.BlockSpec((_E, _D), lambda i: (0, 0)),
        ],
        out_specs=pl.BlockSpec((_TILE, _D), lambda i: (i, 0)),
        out_shape=jax.ShapeDtypeStruct((_R, _D), jnp.float32),
    )(idx3, updated_weight)


def kernel(z, embedding_weight, ema_cluster_size, ema_embed_sum):
    z4 = z.reshape(4, _D, 8, 1, _TILE)
    ecs_col = ema_cluster_size.reshape(_E, 1)

    idx3, updated_weight, loss = _run_stats(
        z4, embedding_weight, ecs_col, ema_embed_sum)

    z_q_flat = _run_sc_gather(updated_weight, idx3.reshape(_R // 128, 128))

    z_q = jnp.transpose(
        z_q_flat[:, :_D].reshape(4, 8, 32, 32, _D), (0, 4, 1, 2, 3))
    indices = idx3.reshape(4, 8, 32, 32)
    return z_q, loss.reshape(()), indices


# speedup vs baseline: 1.0062x; 1.0062x over previous
import functools

import jax
import jax.numpy as jnp
from jax import lax
from jax.experimental import pallas as pl
from jax.experimental.pallas import tpu as pltpu
from jax.experimental.pallas import tpu_sc as plsc

_E = 1024
_D = 64
_R = 32768
_TILE = 1024
_NT = _R // _TILE
_DECAY = 0.99
_EPS = 1e-05
_COMMIT = 0.25


def _stats_body(z_ref, w_ref, ecs_ref, ees_ref,
                idx_ref, uw_ref, loss_ref,
                cnt_acc, sum_acc, z2_acc, wneg_acc, wh2_acc):
    step = pl.program_id(0)

    @pl.when(step == 0)
    def _init():
        cnt_acc[...] = jnp.zeros_like(cnt_acc)
        sum_acc[...] = jnp.zeros_like(sum_acc)
        z2_acc[...] = jnp.zeros_like(z2_acc)
        w0 = w_ref[...]
        wneg_acc[...] = -w0
        wh2_acc[...] = 0.5 * jnp.sum(w0 * w0, axis=1, keepdims=True)

    rowid = lax.broadcasted_iota(jnp.int32, (_E, _TILE), 0)
    wneg = wneg_acc[...]
    wh2 = wh2_acc[...]
    for t in range(4):
        zT = z_ref[0, :, t, 0, :]
        distT = (lax.dot_general(wneg, zT, (((1,), (0,)), ((), ())))
                 + wh2)
        idx = jnp.argmin(distT, axis=0).astype(jnp.int32)
        idx_ref[0, t, :] = idx

        onehotT = (rowid == idx[None, :]).astype(jnp.float32)
        cnt_acc[...] += jnp.sum(onehotT, axis=1, keepdims=True)
        sum_acc[...] += lax.dot_general(onehotT, zT, (((1,), (1,)), ((), ())))
        z2_acc[...] += jnp.sum(zT * zT).reshape(1, 1)

    @pl.when(step == _NT // 4 - 1)
    def _fin():
        cnt = cnt_acc[...]
        new_cs = _DECAY * ecs_ref[...] + (1.0 - _DECAY) * cnt
        new_es = _DECAY * ees_ref[...] + (1.0 - _DECAY) * sum_acc[...]
        n = jnp.sum(new_cs)
        cs = (new_cs + _EPS) / (n + _E * _EPS) * n
        w_new = new_es / cs
        uw_ref[...] = jnp.concatenate(
            [w_new, jnp.zeros((_E, 128 - _D), jnp.float32)], axis=1)
        s_dot_w = jnp.sum(sum_acc[...] * w_new)
        c_w2 = jnp.sum(cnt * jnp.sum(w_new * w_new, axis=1, keepdims=True))
        total = z2_acc[...] - 2.0 * s_dot_w + c_w2
        loss_ref[...] = _COMMIT * total / float(_R * _D)


def _run_stats(z4, embedding_weight, ecs_col, ema_embed_sum):
    return pl.pallas_call(
        _stats_body,
        grid=(_NT // 4,),
        in_specs=[
            pl.BlockSpec((1, _D, 4, 1, _TILE), lambda i: (i // 2, 0, i % 2, 0, 0)),
            pl.BlockSpec((_E, _D), lambda i: (0, 0)),
            pl.BlockSpec((_E, 1), lambda i: (0, 0)),
            pl.BlockSpec((_E, _D), lambda i: (0, 0)),
        ],
        out_specs=[
            pl.BlockSpec((1, 4, _TILE), lambda i: (i, 0, 0)),
            pl.BlockSpec((_E, 128), lambda i: (0, 0)),
            pl.BlockSpec((1, 1), lambda i: (0, 0)),
        ],
        out_shape=[
            jax.ShapeDtypeStruct((_NT // 4, 4, _TILE), jnp.int32),
            jax.ShapeDtypeStruct((_E, 128), jnp.float32),
            jax.ShapeDtypeStruct((1, 1), jnp.float32),
        ],
        scratch_shapes=[
            pltpu.VMEM((_E, 1), jnp.float32),
            pltpu.VMEM((_E, _D), jnp.float32),
            pltpu.VMEM((1, 1), jnp.float32),
            pltpu.VMEM((_E, _D), jnp.float32),
            pltpu.VMEM((_E, 1), jnp.float32),
        ],
    )(z4, embedding_weight, ecs_col, ema_embed_sum)


def _run_sc_gather(table, idx_flat):
    info = plsc.get_sparse_core_info()
    nw = info.num_cores * info.num_subcores
    b_per_w = _R // nw
    n_chunks = b_per_w // 128
    half = n_chunks // 2
    mesh = plsc.VectorSubcoreMesh(core_axis_name="c", subcore_axis_name="s")

    @functools.partial(
        pl.kernel, mesh=mesh,
        out_type=jax.ShapeDtypeStruct((_R, 128), jnp.float32),
        scratch_types=[
            pltpu.VMEM((n_chunks, 128), jnp.int32),
            pltpu.VMEM((half * 128, 128), jnp.float32),
            pltpu.SemaphoreType.DMA,
        ],
    )
    def k(table_hbm, idx_hbm, out_hbm, idx_v, rows_v, sem):
        wid = lax.axis_index("s") * info.num_cores + lax.axis_index("c")
        base = wid * b_per_w
        pltpu.sync_copy(idx_hbm.at[pl.ds(wid * n_chunks, n_chunks)], idx_v)
        for h in range(2):
            copies = []
            for j in range(half):
                copies.append(pltpu.async_copy(
                    table_hbm.at[idx_v.at[h * half + j]],
                    rows_v.at[pl.ds(j * 128, 128)], sem))
            for c in copies:
                c.wait()
            pltpu.sync_copy(
                rows_v, out_hbm.at[pl.ds(base + h * half * 128, half * 128)])

    return k(table, idx_flat)


def kernel(z, embedding_weight, ema_cluster_size, ema_embed_sum):
    z4 = z.reshape(4, _D, 8, 1, _TILE)
    ecs_col = ema_cluster_size.reshape(_E, 1)

    idx3, updated_weight, loss = _run_stats(
        z4, embedding_weight, ecs_col, ema_embed_sum)

    z_q_flat = _run_sc_gather(updated_weight, idx3.reshape(_R // 128, 128))

    z_q = jnp.transpose(
        z_q_flat[:, :_D].reshape(4, 8, 32, 32, _D), (0, 4, 1, 2, 3))
    indices = idx3.reshape(4, 8, 32, 32)
    return z_q, loss.reshape(()), indices
